# Initial kernel scaffold; baseline (speedup 1.0000x reference)
import functools

import jax
import jax.numpy as jnp
from jax import lax
from jax.experimental import pallas as pl
from jax.experimental.pallas import tpu as pltpu
from jax.experimental.pallas import tpu_sc as plsc

_N_VARS = 100000
_N_CLAUSES = 100000
_E = 1600000
_P = 3.0
_A = 10.0

_NCORES = 2
_NSUB = 16
_NW = _NCORES * _NSUB
_L = 16

_C = 2048
_K = _C // 128
_NCHUNK = 25
_T = _C * _NCHUNK
_E_PAD = _T * _NW
_ROWS_T = _T // 128

_NC_PAD = 100096
_SL = _NC_PAD // _NSUB
_R = _NC_PAD // 128


def _sc_kernel_body(xv_hbm, dstp, srcp, dstn, srcn,
                    num0, den0, num1, den1,
                    xv_v, dst_v, src_v, num_v, den_v, stage_v,
                    acc_num, acc_den, sem):
    cid = lax.axis_index("c")
    sid = lax.axis_index("s")
    wid = sid * _NCORES + cid

    @pl.loop(0, _SL // _L)
    def _zero(j):
        stage_v[pl.ds(j * _L, _L)] = jnp.zeros((_L,), jnp.float32)

    pltpu.sync_copy(stage_v, acc_num.at[pl.ds(sid * _SL, _SL)])
    pltpu.sync_copy(stage_v, acc_den.at[pl.ds(sid * _SL, _SL)])

    pltpu.sync_copy(xv_hbm, xv_v)

    plsc.subcore_barrier()

    def process(dst2d, src1d, is_neg):
        base = wid * _T
        rbase = wid * _ROWS_T

        @pl.loop(0, _NCHUNK)
        def _chunk(i):
            pltpu.sync_copy(dst2d.at[pl.ds(rbase + i * _K, _K)], dst_v)
            pltpu.sync_copy(src1d.at[pl.ds(base + i * _C, _C)], src_v)

            @plsc.parallel_loop(0, _C // _L, unroll=8)
            def _vec(j):
                idx = src_v[pl.ds(j * _L, _L)]
                v = plsc.load_gather(xv_v, [idx])
                if is_neg:
                    v = 1.0 - v
                e = jnp.exp(_P * v)
                num_v[pl.ds(j * _L, _L)] = v * e
                den_v[pl.ds(j * _L, _L)] = e

            copies = []
            for j in range(_K):
                copies.append(pltpu.async_copy(
                    num_v.at[pl.ds(j * 128, 128)],
                    acc_num.at[dst_v.at[j]], sem, add=True))
                copies.append(pltpu.async_copy(
                    den_v.at[pl.ds(j * 128, 128)],
                    acc_den.at[dst_v.at[j]], sem, add=True))
            for c in copies:
                c.wait()

    process(dstp, srcp, False)
    process(dstn, srcn, True)

    plsc.subcore_barrier()

    num_out = [num0, num1]
    den_out = [den0, den1]
    for core in range(_NCORES):
        @pl.when(cid == core)
        def _():
            sl = pl.ds(sid * _SL, _SL)
            pltpu.sync_copy(acc_num.at[sl], stage_v)
            pltpu.sync_copy(stage_v, num_out[core].at[sl])
            pltpu.sync_copy(acc_den.at[sl], stage_v)
            pltpu.sync_copy(stage_v, den_out[core].at[sl])


def _make_sc_kernel():
    mesh = plsc.VectorSubcoreMesh(
        core_axis_name="c", subcore_axis_name="s",
        num_cores=_NCORES, num_subcores=_NSUB)
    out = jax.ShapeDtypeStruct((_NC_PAD,), jnp.float32)
    return pl.kernel(
        _sc_kernel_body,
        out_type=(out, out, out, out),
        mesh=mesh,
        scratch_types=[
            pltpu.VMEM((_N_VARS,), jnp.float32),
            pltpu.VMEM((_K, 128), jnp.int32),
            pltpu.VMEM((_C,), jnp.int32),
            pltpu.VMEM((_C,), jnp.float32),
            pltpu.VMEM((_C,), jnp.float32),
            pltpu.VMEM((_SL,), jnp.float32),
            pltpu.VMEM_SHARED((_NC_PAD,), jnp.float32),
            pltpu.VMEM_SHARED((_NC_PAD,), jnp.float32),
            pltpu.SemaphoreType.DMA,
        ],
    )


def _fin_body(n0_ref, n1_ref, d0_ref, d1_ref, cc_ref, out_ref):
    num = n0_ref[...] + n1_ref[...]
    den = d0_ref[...] + d1_ref[...]
    sm = 1.0 / (1.0 + jnp.exp(_A * (0.5 - num / den)))
    row = lax.broadcasted_iota(jnp.int32, (_R, 128), 0)
    col = lax.broadcasted_iota(jnp.int32, (_R, 128), 1)
    mask = (row * 128 + col) < _N_CLAUSES
    diff = jnp.where(mask, sm - cc_ref[...], 0.0)
    out_ref[0, 0] = jnp.sum(diff * diff) / _N_CLAUSES


def _finalize(n0, n1, d0, d1, cc_pad):
    shape2d = (_R, 128)
    loss = pl.pallas_call(
        _fin_body,
        out_shape=jax.ShapeDtypeStruct((1, 1), jnp.float32),
        in_specs=[pl.BlockSpec(memory_space=pltpu.VMEM)] * 5,
        out_specs=pl.BlockSpec(memory_space=pltpu.SMEM),
    )(n0.reshape(shape2d), n1.reshape(shape2d),
      d0.reshape(shape2d), d1.reshape(shape2d), cc_pad.reshape(shape2d))
    return loss[0, 0]


def kernel(xv, adj_pos, adj_neg, clause_count, is_train):
    del is_train
    xvf = xv.reshape(-1)
    padn = _E_PAD - _E
    pad_dst = jnp.full((padn,), _N_CLAUSES, jnp.int32)
    pad_src = jnp.zeros((padn,), jnp.int32)

    dstp = jnp.concatenate([adj_pos[0], pad_dst]).reshape(_E_PAD // 128, 128)
    srcp = jnp.concatenate([adj_pos[1], pad_src])
    dstn = jnp.concatenate([adj_neg[0], pad_dst]).reshape(_E_PAD // 128, 128)
    srcn = jnp.concatenate([adj_neg[1], pad_src])

    n0, d0, n1, d1 = _make_sc_kernel()(xvf, dstp, srcp, dstn, srcn)

    cc_pad = jnp.pad(clause_count, (0, _NC_PAD - _N_CLAUSES))
    return _finalize(n0, n1, d0, d1, cc_pad)


# speedup vs baseline: 117.9750x; 117.9750x over previous
import functools

import jax
import jax.numpy as jnp
from jax import lax
from jax.experimental import pallas as pl
from jax.experimental.pallas import tpu as pltpu
from jax.experimental.pallas import tpu_sc as plsc

_N_VARS = 100000
_N_CLAUSES = 100000
_E = 1600000
_P = 3.0
_A = 10.0

_NCORES = 2
_NSUB = 16
_NW = _NCORES * _NSUB
_L = 16

_C = 2048
_K = _C // 128
_NCHUNK = 25
_T = _C * _NCHUNK
_E_PAD = _T * _NW
_ROWS_T = _T // 128

_NC_PAD = 100096
_SL = _NC_PAD // _NSUB
_R = _NC_PAD // 128


def _sc_kernel_body(xv_hbm, dstp, srcp, dstn, srcn,
                    num0, den0, num1, den1,
                    xv_v, dst_v, src_v, num_v, den_v, stage_v,
                    acc_num, acc_den, sem):
    cid = lax.axis_index("c")
    sid = lax.axis_index("s")
    wid = sid * _NCORES + cid

    @pl.loop(0, _SL // _L)
    def _zero(j):
        stage_v[pl.ds(j * _L, _L)] = jnp.zeros((_L,), jnp.float32)

    pltpu.sync_copy(stage_v, acc_num.at[pl.ds(sid * _SL, _SL)])
    pltpu.sync_copy(stage_v, acc_den.at[pl.ds(sid * _SL, _SL)])

    pltpu.sync_copy(xv_hbm, xv_v)

    plsc.subcore_barrier()

    def process(dst2d, src1d, is_neg):
        base = wid * _T
        rbase = wid * _ROWS_T

        @pl.loop(0, _NCHUNK)
        def _chunk(i):
            pltpu.sync_copy(dst2d.at[pl.ds(rbase + i * _K, _K)], dst_v)
            pltpu.sync_copy(src1d.at[pl.ds(base + i * _C, _C)], src_v)

            @plsc.parallel_loop(0, _C // _L, unroll=8)
            def _vec(j):
                idx = src_v[pl.ds(j * _L, _L)]
                v = plsc.load_gather(xv_v, [idx])
                if is_neg:
                    v = 1.0 - v
                e = jnp.exp(_P * v)
                num_v[pl.ds(j * _L, _L)] = v * e
                den_v[pl.ds(j * _L, _L)] = e

            copies = []
            for j in range(_K):
                copies.append(pltpu.async_copy(
                    num_v.at[pl.ds(j * 128, 128)],
                    acc_num.at[dst_v.at[j]], sem, add=True))
                copies.append(pltpu.async_copy(
                    den_v.at[pl.ds(j * 128, 128)],
                    acc_den.at[dst_v.at[j]], sem, add=True))
            for c in copies:
                c.wait()

    process(dstp, srcp, False)
    process(dstn, srcn, True)

    plsc.subcore_barrier()

    num_out = [num0, num1]
    den_out = [den0, den1]
    for core in range(_NCORES):
        @pl.when(cid == core)
        def _():
            sl = pl.ds(sid * _SL, _SL)
            pltpu.sync_copy(acc_num.at[sl], stage_v)
            pltpu.sync_copy(stage_v, num_out[core].at[sl])
            pltpu.sync_copy(acc_den.at[sl], stage_v)
            pltpu.sync_copy(stage_v, den_out[core].at[sl])


def _make_sc_kernel():
    mesh = plsc.VectorSubcoreMesh(
        core_axis_name="c", subcore_axis_name="s",
        num_cores=_NCORES, num_subcores=_NSUB)
    out = jax.ShapeDtypeStruct((_NC_PAD,), jnp.float32)
    return pl.kernel(
        _sc_kernel_body,
        out_type=(out, out, out, out),
        mesh=mesh,
        compiler_params=pltpu.CompilerParams(needs_layout_passes=False),
        scratch_types=[
            pltpu.VMEM((_N_VARS,), jnp.float32),
            pltpu.VMEM((_K, 128), jnp.int32),
            pltpu.VMEM((_C,), jnp.int32),
            pltpu.VMEM((_C,), jnp.float32),
            pltpu.VMEM((_C,), jnp.float32),
            pltpu.VMEM((_SL,), jnp.float32),
            pltpu.VMEM_SHARED((_NC_PAD,), jnp.float32),
            pltpu.VMEM_SHARED((_NC_PAD,), jnp.float32),
            pltpu.SemaphoreType.DMA,
        ],
    )


def _fin_body(n0_ref, n1_ref, d0_ref, d1_ref, cc_ref, out_ref):
    num = n0_ref[...] + n1_ref[...]
    den = d0_ref[...] + d1_ref[...]
    sm = 1.0 / (1.0 + jnp.exp(_A * (0.5 - num / den)))
    row = lax.broadcasted_iota(jnp.int32, (_R, 128), 0)
    col = lax.broadcasted_iota(jnp.int32, (_R, 128), 1)
    mask = (row * 128 + col) < _N_CLAUSES
    diff = jnp.where(mask, sm - cc_ref[...], 0.0)
    out_ref[0, 0] = jnp.sum(diff * diff) / _N_CLAUSES


def _finalize(n0, n1, d0, d1, cc_pad):
    shape2d = (_R, 128)
    loss = pl.pallas_call(
        _fin_body,
        out_shape=jax.ShapeDtypeStruct((1, 1), jnp.float32),
        in_specs=[pl.BlockSpec(memory_space=pltpu.VMEM)] * 5,
        out_specs=pl.BlockSpec(memory_space=pltpu.SMEM),
    )(n0.reshape(shape2d), n1.reshape(shape2d),
      d0.reshape(shape2d), d1.reshape(shape2d), cc_pad.reshape(shape2d))
    return loss[0, 0]


def kernel(xv, adj_pos, adj_neg, clause_count, is_train):
    del is_train
    xvf = xv.reshape(-1)
    padn = _E_PAD - _E
    pad_dst = jnp.full((padn,), _N_CLAUSES, jnp.int32)
    pad_src = jnp.zeros((padn,), jnp.int32)

    dstp = jnp.concatenate([adj_pos[0], pad_dst]).reshape(_E_PAD // 128, 128)
    srcp = jnp.concatenate([adj_pos[1], pad_src])
    dstn = jnp.concatenate([adj_neg[0], pad_dst]).reshape(_E_PAD // 128, 128)
    srcn = jnp.concatenate([adj_neg[1], pad_src])

    n0, d0, n1, d1 = _make_sc_kernel()(xvf, dstp, srcp, dstn, srcn)

    cc_pad = jnp.pad(clause_count, (0, _NC_PAD - _N_CLAUSES))
    return _finalize(n0, n1, d0, d1, cc_pad)
